# baseline (speedup 1.0000x reference)
import functools

import jax
import jax.numpy as jnp
from jax import lax
from jax.experimental import pallas as pl
from jax.experimental.pallas import tpu as pltpu
from jax.experimental.pallas import tpu_sc as plsc

EPS = 1e-12
L = 16
BCHUNK = 128
NBUF = 2
BLK = 4

_COMPILER_PARAMS = pltpu.CompilerParams(
    needs_layout_passes=False, use_tc_tiling_on_sc=True
)


def _rsqrt(x):
    xi = lax.bitcast_convert_type(x, jnp.int32)
    yi = jnp.int32(0x5F3759DF) - lax.shift_right_arithmetic(xi, 1)
    y = lax.bitcast_convert_type(yi, jnp.float32)
    for _ in range(3):
        y = y * (1.5 - 0.5 * x * y * y)
    return y


def _block_table(table, mesh):
    V, D = table.shape
    PANEL = BLK * D
    n_full = V // PANEL
    tail = V - n_full * PANEL
    n_tail = tail // BLK
    NW = 32
    tail_blk = table[n_full * PANEL:].reshape(n_tail, BLK * D)

    @functools.partial(
        pl.kernel,
        mesh=mesh,
        compiler_params=_COMPILER_PARAMS,
        out_type=jax.ShapeDtypeStruct((V // BLK, BLK * D), jnp.float32),
        scratch_types=[
            pltpu.VMEM((D, PANEL), jnp.float32),
            pltpu.VMEM((D, PANEL), jnp.float32),
            pltpu.VMEM((D, PANEL), jnp.float32),
            pltpu.VMEM((D, PANEL), jnp.float32),
            pltpu.SemaphoreType.DMA,
            pltpu.SemaphoreType.DMA,
            pltpu.SemaphoreType.DMA,
            pltpu.SemaphoreType.DMA,
        ],
    )
    def tk(tbl_t, tail_hbm, out_hbm, in0, in1, ob0, ob1, gi0, gi1, go0, go1):
        wid = lax.axis_index("s") * 2 + lax.axis_index("c")
        tbuf = [in0, in1]
        obuf = [ob0, ob1]
        isem = [gi0, gi1]
        osem = [go0, go1]
        n_mine = (n_full - wid + NW - 1) // NW

        def panel(t):
            return wid + t * NW

        def issue_in(t, b):
            pltpu.async_copy(
                tbl_t.at[:, pl.ds(panel(t) * PANEL, PANEL)], tbuf[b], isem[b])

        def wait_in(t, b):
            pltpu.make_async_copy(
                tbl_t.at[:, pl.ds(panel(t) * PANEL, PANEL)], tbuf[b],
                isem[b]).wait()

        def issue_out(t, b):
            pltpu.async_copy(
                obuf[b], out_hbm.at[pl.ds(panel(t) * D, D)], osem[b])

        def wait_out(t, b):
            pltpu.make_async_copy(
                obuf[b], out_hbm.at[pl.ds(panel(t) * D, D)], osem[b]).wait()

        def transpose(b, ngroups):
            src, dst = tbuf[b], obuf[b]
            for g in range(ngroups):
                vv = g * L + lax.iota(jnp.int32, L)
                rr = lax.shift_right_logical(vv, 2)
                cb = (vv & 3) * D
                for j in range(D):
                    v = src[j, pl.ds(g * L, L)]
                    plsc.store_scatter(dst, [rr, cb + j], v)

        @pl.when(n_mine > 0)
        def _():
            issue_in(0, 0)

        @pl.when(n_mine > 1)
        def _():
            issue_in(1, 1)

        def head(t, b):
            wait_in(t, b)
            transpose(b, PANEL // L)
            issue_out(t, b)

            @pl.when(t + NBUF < n_mine)
            def _():
                issue_in(t + NBUF, b)

        @pl.when(n_mine > 0)
        def _():
            head(0, 0)

        @pl.when(n_mine > 1)
        def _():
            head(1, 1)

        def steady(i, _):
            for b in range(NBUF):
                t = NBUF + i * NBUF + b

                @pl.when(t < n_mine)
                def _():
                    wait_in(t, b)
                    wait_out(t - NBUF, b)
                    transpose(b, PANEL // L)
                    issue_out(t, b)

                    @pl.when(t + NBUF < n_mine)
                    def _():
                        issue_in(t + NBUF, b)
            return 0

        lax.fori_loop(
            0, jnp.maximum(n_mine - NBUF + 1, 0) // NBUF + 1, steady, 0)

        for b in range(NBUF):
            t_lb = jnp.where((n_mine - 1) % NBUF == b, n_mine - 1, n_mine - 2)

            @pl.when(t_lb >= 0)
            def _():
                wait_out(t_lb, b)

        @pl.when(wid == 0)
        def _():
            pltpu.sync_copy(tail_hbm, in0.at[pl.ds(0, n_tail)])
            pltpu.sync_copy(
                in0.at[pl.ds(0, n_tail)],
                out_hbm.at[pl.ds(n_full * D, n_tail)])

    return tk(table.T, tail_blk)


def kernel(input_tag_ids, table, ln_weight, ln_bias):
    B, S = input_tag_ids.shape
    V, D = table.shape
    NC, NS = 2, 16
    NW = NC * NS
    n_super = S
    assert NW * BCHUNK == B and D == 2 * L and BLK * D == 128

    ids_t = input_tag_ids.T
    mesh = plsc.VectorSubcoreMesh(core_axis_name="c", subcore_axis_name="s")
    tbl_blk = _block_table(table, mesh)

    @functools.partial(
        pl.kernel,
        mesh=mesh,
        compiler_params=_COMPILER_PARAMS,
        out_type=jax.ShapeDtypeStruct((S, D, B), jnp.float32),
        scratch_types=[
            pltpu.VMEM((n_super, BCHUNK), jnp.int32),
            pltpu.VMEM((NBUF, BCHUNK), jnp.int32),
            pltpu.VMEM((BCHUNK, BLK * D), jnp.float32),
            pltpu.VMEM((BCHUNK, BLK * D), jnp.float32),
            pltpu.VMEM((D, BCHUNK), jnp.float32),
            pltpu.VMEM((D, BCHUNK), jnp.float32),
            pltpu.VMEM((D,), jnp.float32),
            pltpu.VMEM((D,), jnp.float32),
            pltpu.SemaphoreType.DMA,
            pltpu.SemaphoreType.DMA,
            pltpu.SemaphoreType.DMA,
            pltpu.SemaphoreType.DMA,
            pltpu.SemaphoreType.DMA,
        ],
    )
    def k(idx_hbm, table_hbm, w_hbm, b_hbm, out_hbm,
          idx_v, blk_v, rows0, rows1, obuf0, obuf1, w_v, b_v,
          g0, g1, o0, o1, isem):
        wid = lax.axis_index("s") * NC + lax.axis_index("c")
        b0 = wid * BCHUNK
        rows = [rows0, rows1]
        obuf = [obuf0, obuf1]
        gsem = [g0, g1]
        osem = [o0, o1]

        def idx_issue(t, _):
            pltpu.async_copy(
                idx_hbm.at[t, pl.ds(b0, BCHUNK)], idx_v.at[t], isem)
            return 0

        def idx_drain(t, _):
            pltpu.make_async_copy(
                idx_hbm.at[t, pl.ds(b0, BCHUNK)], idx_v.at[t], isem).wait()
            return 0

        lax.fori_loop(0, n_super, idx_issue, 0)
        pltpu.sync_copy(w_hbm, w_v)
        pltpu.sync_copy(b_hbm, b_v)
        lax.fori_loop(0, n_super, idx_drain, 0)
        w_lo, w_hi = w_v[pl.ds(0, L)], w_v[pl.ds(L, L)]
        b_lo, b_hi = b_v[pl.ds(0, L)], b_v[pl.ds(L, L)]
        w_sc = [w_lo[c] for c in range(L)] + [w_hi[c] for c in range(L)]
        b_sc = [b_lo[c] for c in range(L)] + [b_hi[c] for c in range(L)]

        def issue_gather(j, b):
            for q in range(BCHUNK // L):
                blk_v[b, pl.ds(q * L, L)] = lax.shift_right_logical(
                    idx_v[j, pl.ds(q * L, L)], 2)
            pltpu.async_copy(
                table_hbm.at[blk_v.at[b]], rows[b], gsem[b])

        def wait_gather(b):
            pltpu.make_async_copy(
                table_hbm.at[blk_v.at[b]], rows[b], gsem[b]).wait()

        def out_slice(j):
            return out_hbm.at[j, :, pl.ds(b0, BCHUNK)]

        def issue_out(j, b):
            pltpu.async_copy(obuf[b], out_slice(j), osem[b])

        def wait_out(j, b):
            pltpu.make_async_copy(obuf[b], out_slice(j), osem[b]).wait()

        def compute(j, b):
            src, dst = rows[b], obuf[b]

            def group_body(g, _):
                t0 = g * L
                rows16 = t0 + lax.iota(jnp.int32, L)
                sub = (idx_v[j, pl.ds(t0, L)] & 3) * D
                cols = []
                part = [jnp.zeros((L,), jnp.float32) for _ in range(4)]
                for c in range(D):
                    v = plsc.load_gather(src, [rows16, sub + c])
                    cols.append(v)
                    part[c % 4] = part[c % 4] + v
                u = ((part[0] + part[1]) + (part[2] + part[3])) * (1.0 / D)
                part = [jnp.zeros((L,), jnp.float32) for _ in range(4)]
                for c in range(D):
                    cols[c] = cols[c] - u
                    part[c % 4] = part[c % 4] + cols[c] * cols[c]
                s2 = (part[0] + part[1]) + (part[2] + part[3])
                inv = _rsqrt(jnp.maximum(s2 * (1.0 / D), 0.0) + EPS)
                for c in range(D):
                    dst[c, pl.ds(t0, L)] = cols[c] * inv * w_sc[c] + b_sc[c]
                return 0

            lax.fori_loop(0, BCHUNK // L, group_body, 0)

        issue_gather(0, 0)
        issue_gather(1, 1)

        for j in range(NBUF):
            b = j % NBUF
            wait_gather(b)
            compute(j, b)
            issue_out(j, b)
            issue_gather(j + NBUF, b)

        def steady(i, _):
            j0 = NBUF + i * NBUF
            for b in range(NBUF):
                j = j0 + b
                wait_gather(b)
                wait_out(j - NBUF, b)
                compute(j, b)
                issue_out(j, b)
                issue_gather(j + NBUF, b)
            return 0

        lax.fori_loop(0, (n_super - 2 * NBUF) // NBUF, steady, 0)

        for j in range(n_super - NBUF, n_super):
            b = j % NBUF
            wait_gather(b)
            wait_out(j - NBUF, b)
            compute(j, b)
            issue_out(j, b)
        for j in range(n_super - NBUF, n_super):
            wait_out(j, j % NBUF)

    out = k(ids_t, tbl_blk, ln_weight, ln_bias)
    return jnp.transpose(out, (2, 0, 1))


# speedup vs baseline: 1.3482x; 1.3482x over previous
import functools

import jax
import jax.numpy as jnp
from jax import lax
from jax.experimental import pallas as pl
from jax.experimental.pallas import tpu as pltpu
from jax.experimental.pallas import tpu_sc as plsc

EPS = 1e-12
L = 16
BCHUNK = 128
NBUF = 2
BLK = 4

_COMPILER_PARAMS = pltpu.CompilerParams(
    needs_layout_passes=False, use_tc_tiling_on_sc=True
)


def _rsqrt(x):
    xi = lax.bitcast_convert_type(x, jnp.int32)
    yi = jnp.int32(0x5F3759DF) - lax.shift_right_arithmetic(xi, 1)
    y = lax.bitcast_convert_type(yi, jnp.float32)
    for _ in range(3):
        y = y * (1.5 - 0.5 * x * y * y)
    return y


def _block_table(table, mesh):
    V, D = table.shape
    PANEL = BLK * D
    n_full = V // PANEL
    tail = V - n_full * PANEL
    n_tail = tail // BLK
    NW = 32
    tail_blk = table[n_full * PANEL:].reshape(n_tail, BLK * D)

    @functools.partial(
        pl.kernel,
        mesh=mesh,
        compiler_params=_COMPILER_PARAMS,
        out_type=jax.ShapeDtypeStruct((V // BLK, BLK * D), jnp.float32),
        scratch_types=[
            pltpu.VMEM((D, PANEL), jnp.float32),
            pltpu.VMEM((D, PANEL), jnp.float32),
            pltpu.VMEM((D, PANEL), jnp.float32),
            pltpu.VMEM((D, PANEL), jnp.float32),
            pltpu.SemaphoreType.DMA,
            pltpu.SemaphoreType.DMA,
            pltpu.SemaphoreType.DMA,
            pltpu.SemaphoreType.DMA,
        ],
    )
    def tk(tbl_t, tail_hbm, out_hbm, in0, in1, ob0, ob1, gi0, gi1, go0, go1):
        wid = lax.axis_index("s") * 2 + lax.axis_index("c")
        tbuf = [in0, in1]
        obuf = [ob0, ob1]
        isem = [gi0, gi1]
        osem = [go0, go1]
        n_mine = (n_full - wid + NW - 1) // NW

        def panel(t):
            return wid + t * NW

        def issue_in(t, b):
            pltpu.async_copy(
                tbl_t.at[:, pl.ds(panel(t) * PANEL, PANEL)], tbuf[b], isem[b])

        def wait_in(t, b):
            pltpu.make_async_copy(
                tbl_t.at[:, pl.ds(panel(t) * PANEL, PANEL)], tbuf[b],
                isem[b]).wait()

        def issue_out(t, b):
            pltpu.async_copy(
                obuf[b], out_hbm.at[pl.ds(panel(t) * D, D)], osem[b])

        def wait_out(t, b):
            pltpu.make_async_copy(
                obuf[b], out_hbm.at[pl.ds(panel(t) * D, D)], osem[b]).wait()

        def transpose(b, ngroups):
            src, dst = tbuf[b], obuf[b]
            lane = lax.iota(jnp.int32, L)
            for g in range(ngroups):
                vv = g * L + lane
                rr = lax.shift_right_logical(vv, 2)
                cb = (vv & 3) * D
                for m in range(D):
                    jl = (m + lane) & (D - 1)
                    v = plsc.load_gather(src, [jl, vv])
                    plsc.store_scatter(dst, [rr, cb + jl], v)

        @pl.when(n_mine > 0)
        def _():
            issue_in(0, 0)

        @pl.when(n_mine > 1)
        def _():
            issue_in(1, 1)

        def head(t, b):
            wait_in(t, b)
            transpose(b, PANEL // L)
            issue_out(t, b)

            @pl.when(t + NBUF < n_mine)
            def _():
                issue_in(t + NBUF, b)

        @pl.when(n_mine > 0)
        def _():
            head(0, 0)

        @pl.when(n_mine > 1)
        def _():
            head(1, 1)

        def steady(i, _):
            for b in range(NBUF):
                t = NBUF + i * NBUF + b

                @pl.when(t < n_mine)
                def _():
                    wait_in(t, b)
                    wait_out(t - NBUF, b)
                    transpose(b, PANEL // L)
                    issue_out(t, b)

                    @pl.when(t + NBUF < n_mine)
                    def _():
                        issue_in(t + NBUF, b)
            return 0

        lax.fori_loop(
            0, jnp.maximum(n_mine - NBUF + 1, 0) // NBUF + 1, steady, 0)

        for b in range(NBUF):
            t_lb = jnp.where((n_mine - 1) % NBUF == b, n_mine - 1, n_mine - 2)

            @pl.when(t_lb >= 0)
            def _():
                wait_out(t_lb, b)

        @pl.when(wid == 0)
        def _():
            pltpu.sync_copy(tail_hbm, in0.at[pl.ds(0, n_tail)])
            pltpu.sync_copy(
                in0.at[pl.ds(0, n_tail)],
                out_hbm.at[pl.ds(n_full * D, n_tail)])

    return tk(table.T, tail_blk)


def kernel(input_tag_ids, table, ln_weight, ln_bias):
    B, S = input_tag_ids.shape
    V, D = table.shape
    NC, NS = 2, 16
    NW = NC * NS
    n_super = S
    assert NW * BCHUNK == B and D == 2 * L and BLK * D == 128

    ids_t = input_tag_ids.T
    mesh = plsc.VectorSubcoreMesh(core_axis_name="c", subcore_axis_name="s")
    tbl_blk = _block_table(table, mesh)

    @functools.partial(
        pl.kernel,
        mesh=mesh,
        compiler_params=_COMPILER_PARAMS,
        out_type=jax.ShapeDtypeStruct((S, D, B), jnp.float32),
        scratch_types=[
            pltpu.VMEM((n_super, BCHUNK), jnp.int32),
            pltpu.VMEM((NBUF, BCHUNK), jnp.int32),
            pltpu.VMEM((BCHUNK, BLK * D), jnp.float32),
            pltpu.VMEM((BCHUNK, BLK * D), jnp.float32),
            pltpu.VMEM((D, BCHUNK), jnp.float32),
            pltpu.VMEM((D, BCHUNK), jnp.float32),
            pltpu.VMEM((D,), jnp.float32),
            pltpu.VMEM((D,), jnp.float32),
            pltpu.SemaphoreType.DMA,
            pltpu.SemaphoreType.DMA,
            pltpu.SemaphoreType.DMA,
            pltpu.SemaphoreType.DMA,
            pltpu.SemaphoreType.DMA,
        ],
    )
    def k(idx_hbm, table_hbm, w_hbm, b_hbm, out_hbm,
          idx_v, blk_v, rows0, rows1, obuf0, obuf1, w_v, b_v,
          g0, g1, o0, o1, isem):
        wid = lax.axis_index("s") * NC + lax.axis_index("c")
        b0 = wid * BCHUNK
        rows = [rows0, rows1]
        obuf = [obuf0, obuf1]
        gsem = [g0, g1]
        osem = [o0, o1]

        def idx_issue(t, _):
            pltpu.async_copy(
                idx_hbm.at[t, pl.ds(b0, BCHUNK)], idx_v.at[t], isem)
            return 0

        def idx_drain(t, _):
            pltpu.make_async_copy(
                idx_hbm.at[t, pl.ds(b0, BCHUNK)], idx_v.at[t], isem).wait()
            return 0

        lax.fori_loop(0, n_super, idx_issue, 0)
        pltpu.sync_copy(w_hbm, w_v)
        pltpu.sync_copy(b_hbm, b_v)
        lax.fori_loop(0, n_super, idx_drain, 0)
        w_lo, w_hi = w_v[pl.ds(0, L)], w_v[pl.ds(L, L)]
        b_lo, b_hi = b_v[pl.ds(0, L)], b_v[pl.ds(L, L)]
        w_sc = [w_lo[c] for c in range(L)] + [w_hi[c] for c in range(L)]
        b_sc = [b_lo[c] for c in range(L)] + [b_hi[c] for c in range(L)]

        def issue_gather(j, b):
            for q in range(BCHUNK // L):
                blk_v[b, pl.ds(q * L, L)] = lax.shift_right_logical(
                    idx_v[j, pl.ds(q * L, L)], 2)
            pltpu.async_copy(
                table_hbm.at[blk_v.at[b]], rows[b], gsem[b])

        def wait_gather(b):
            pltpu.make_async_copy(
                table_hbm.at[blk_v.at[b]], rows[b], gsem[b]).wait()

        def out_slice(j):
            return out_hbm.at[j, :, pl.ds(b0, BCHUNK)]

        def issue_out(j, b):
            pltpu.async_copy(obuf[b], out_slice(j), osem[b])

        def wait_out(j, b):
            pltpu.make_async_copy(obuf[b], out_slice(j), osem[b]).wait()

        def compute(j, b):
            src, dst = rows[b], obuf[b]

            def group_body(g, _):
                t0 = g * L
                lane = lax.iota(jnp.int32, L)
                toks = t0 + lane
                sub = (idx_v[j, pl.ds(t0, L)] & 3) * D
                diag = []
                part = [jnp.zeros((L,), jnp.float32) for _ in range(4)]
                for m in range(D):
                    cl = (m + lane) & (D - 1)
                    v = plsc.load_gather(src, [toks, sub + cl])
                    diag.append(v)
                    part[m % 4] = part[m % 4] + v
                u = ((part[0] + part[1]) + (part[2] + part[3])) * (1.0 / D)
                part = [jnp.zeros((L,), jnp.float32) for _ in range(4)]
                for m in range(D):
                    diag[m] = diag[m] - u
                    part[m % 4] = part[m % 4] + diag[m] * diag[m]
                s2 = (part[0] + part[1]) + (part[2] + part[3])
                inv = _rsqrt(jnp.maximum(s2 * (1.0 / D), 0.0) + EPS)
                for m in range(D):
                    cl = (m + lane) & (D - 1)
                    plsc.store_scatter(dst, [cl, toks], diag[m] * inv)
                for c in range(D):
                    x = dst[c, pl.ds(t0, L)]
                    dst[c, pl.ds(t0, L)] = x * w_sc[c] + b_sc[c]
                return 0

            lax.fori_loop(0, BCHUNK // L, group_body, 0)

        issue_gather(0, 0)
        issue_gather(1, 1)

        for j in range(NBUF):
            b = j % NBUF
            wait_gather(b)
            compute(j, b)
            issue_out(j, b)
            issue_gather(j + NBUF, b)

        def steady(i, _):
            j0 = NBUF + i * NBUF
            for b in range(NBUF):
                j = j0 + b
                wait_gather(b)
                wait_out(j - NBUF, b)
                compute(j, b)
                issue_out(j, b)
                issue_gather(j + NBUF, b)
            return 0

        lax.fori_loop(0, (n_super - 2 * NBUF) // NBUF, steady, 0)

        for j in range(n_super - NBUF, n_super):
            b = j % NBUF
            wait_gather(b)
            wait_out(j - NBUF, b)
            compute(j, b)
            issue_out(j, b)
        for j in range(n_super - NBUF, n_super):
            wait_out(j, j % NBUF)

    out = k(ids_t, tbl_blk, ln_weight, ln_bias)
    return jnp.transpose(out, (2, 0, 1))


# speedup vs baseline: 1.5265x; 1.1323x over previous
import functools

import jax
import jax.numpy as jnp
from jax import lax
from jax.experimental import pallas as pl
from jax.experimental.pallas import tpu as pltpu
from jax.experimental.pallas import tpu_sc as plsc

EPS = 1e-12
L = 16
BCHUNK = 128
NBUF = 2
BLK = 4

_COMPILER_PARAMS = pltpu.CompilerParams(
    needs_layout_passes=False, use_tc_tiling_on_sc=True
)


def _rsqrt(x):
    xi = lax.bitcast_convert_type(x, jnp.int32)
    yi = jnp.int32(0x5F3759DF) - lax.shift_right_arithmetic(xi, 1)
    y = lax.bitcast_convert_type(yi, jnp.float32)
    for _ in range(3):
        y = y * (1.5 - 0.5 * x * y * y)
    return y


def _block_table(table, mesh):
    V, D = table.shape
    PANEL = BLK * D
    n_full = V // PANEL
    tail = V - n_full * PANEL
    n_tail = tail // BLK
    NW = 32
    tail_blk = table[n_full * PANEL:].reshape(n_tail, BLK * D)

    @functools.partial(
        pl.kernel,
        mesh=mesh,
        compiler_params=_COMPILER_PARAMS,
        out_type=jax.ShapeDtypeStruct((V // BLK, BLK * D), jnp.float32),
        scratch_types=[
            pltpu.VMEM((D, PANEL), jnp.float32),
            pltpu.VMEM((D, PANEL), jnp.float32),
            pltpu.VMEM((D, PANEL), jnp.float32),
            pltpu.VMEM((D, PANEL), jnp.float32),
            pltpu.SemaphoreType.DMA,
            pltpu.SemaphoreType.DMA,
            pltpu.SemaphoreType.DMA,
            pltpu.SemaphoreType.DMA,
        ],
    )
    def tk(tbl_t, tail_hbm, out_hbm, in0, in1, ob0, ob1, gi0, gi1, go0, go1):
        wid = lax.axis_index("s") * 2 + lax.axis_index("c")
        tbuf = [in0, in1]
        obuf = [ob0, ob1]
        isem = [gi0, gi1]
        osem = [go0, go1]
        n_mine = (n_full - wid + NW - 1) // NW

        def panel(t):
            return wid + t * NW

        def issue_in(t, b):
            pltpu.async_copy(
                tbl_t.at[:, pl.ds(panel(t) * PANEL, PANEL)], tbuf[b], isem[b])

        def wait_in(t, b):
            pltpu.make_async_copy(
                tbl_t.at[:, pl.ds(panel(t) * PANEL, PANEL)], tbuf[b],
                isem[b]).wait()

        def issue_out(t, b):
            pltpu.async_copy(
                obuf[b], out_hbm.at[pl.ds(panel(t) * D, D)], osem[b])

        def wait_out(t, b):
            pltpu.make_async_copy(
                obuf[b], out_hbm.at[pl.ds(panel(t) * D, D)], osem[b]).wait()

        def transpose(b, ngroups):
            src, dst = tbuf[b], obuf[b]
            lane = lax.iota(jnp.int32, L)
            for g in range(ngroups):
                vv = g * L + lane
                rr = lax.shift_right_logical(vv, 2)
                cb = (vv & 3) * D
                vals = [
                    plsc.load_gather(src, [(m + lane) & (D - 1), vv])
                    for m in range(D)
                ]
                for m in range(D):
                    jl = (m + lane) & (D - 1)
                    plsc.store_scatter(dst, [rr, cb + jl], vals[m])

        @pl.when(n_mine > 0)
        def _():
            issue_in(0, 0)

        @pl.when(n_mine > 1)
        def _():
            issue_in(1, 1)

        def head(t, b):
            wait_in(t, b)
            transpose(b, PANEL // L)
            issue_out(t, b)

            @pl.when(t + NBUF < n_mine)
            def _():
                issue_in(t + NBUF, b)

        @pl.when(n_mine > 0)
        def _():
            head(0, 0)

        @pl.when(n_mine > 1)
        def _():
            head(1, 1)

        def steady(i, _):
            for b in range(NBUF):
                t = NBUF + i * NBUF + b

                @pl.when(t < n_mine)
                def _():
                    wait_in(t, b)
                    wait_out(t - NBUF, b)
                    transpose(b, PANEL // L)
                    issue_out(t, b)

                    @pl.when(t + NBUF < n_mine)
                    def _():
                        issue_in(t + NBUF, b)
            return 0

        lax.fori_loop(
            0, jnp.maximum(n_mine - NBUF + 1, 0) // NBUF + 1, steady, 0)

        for b in range(NBUF):
            t_lb = jnp.where((n_mine - 1) % NBUF == b, n_mine - 1, n_mine - 2)

            @pl.when(t_lb >= 0)
            def _():
                wait_out(t_lb, b)

        @pl.when(wid == 0)
        def _():
            pltpu.sync_copy(tail_hbm, in0.at[pl.ds(0, n_tail)])
            pltpu.sync_copy(
                in0.at[pl.ds(0, n_tail)],
                out_hbm.at[pl.ds(n_full * D, n_tail)])

    return tk(table.T, tail_blk)


def kernel(input_tag_ids, table, ln_weight, ln_bias):
    B, S = input_tag_ids.shape
    V, D = table.shape
    NC, NS = 2, 16
    NW = NC * NS
    n_super = S
    assert NW * BCHUNK == B and D == 2 * L and BLK * D == 128

    ids_t = input_tag_ids.T
    mesh = plsc.VectorSubcoreMesh(core_axis_name="c", subcore_axis_name="s")
    tbl_blk = _block_table(table, mesh)

    @functools.partial(
        pl.kernel,
        mesh=mesh,
        compiler_params=_COMPILER_PARAMS,
        out_type=jax.ShapeDtypeStruct((S, D, B), jnp.float32),
        scratch_types=[
            pltpu.VMEM((n_super, BCHUNK), jnp.int32),
            pltpu.VMEM((NBUF, BCHUNK), jnp.int32),
            pltpu.VMEM((BCHUNK, BLK * D), jnp.float32),
            pltpu.VMEM((BCHUNK, BLK * D), jnp.float32),
            pltpu.VMEM((D, BCHUNK), jnp.float32),
            pltpu.VMEM((D, BCHUNK), jnp.float32),
            pltpu.VMEM((D,), jnp.float32),
            pltpu.VMEM((D,), jnp.float32),
            pltpu.SemaphoreType.DMA,
            pltpu.SemaphoreType.DMA,
            pltpu.SemaphoreType.DMA,
            pltpu.SemaphoreType.DMA,
            pltpu.SemaphoreType.DMA,
        ],
    )
    def k(idx_hbm, table_hbm, w_hbm, b_hbm, out_hbm,
          idx_v, blk_v, rows0, rows1, obuf0, obuf1, w_v, b_v,
          g0, g1, o0, o1, isem):
        wid = lax.axis_index("s") * NC + lax.axis_index("c")
        b0 = wid * BCHUNK
        rows = [rows0, rows1]
        obuf = [obuf0, obuf1]
        gsem = [g0, g1]
        osem = [o0, o1]

        def idx_issue(t, _):
            pltpu.async_copy(
                idx_hbm.at[t, pl.ds(b0, BCHUNK)], idx_v.at[t], isem)
            return 0

        def idx_drain(t, _):
            pltpu.make_async_copy(
                idx_hbm.at[t, pl.ds(b0, BCHUNK)], idx_v.at[t], isem).wait()
            return 0

        lax.fori_loop(0, n_super, idx_issue, 0)
        pltpu.sync_copy(w_hbm, w_v)
        pltpu.sync_copy(b_hbm, b_v)
        lax.fori_loop(0, n_super, idx_drain, 0)
        w_lo, w_hi = w_v[pl.ds(0, L)], w_v[pl.ds(L, L)]
        b_lo, b_hi = b_v[pl.ds(0, L)], b_v[pl.ds(L, L)]
        w_sc = [w_lo[c] for c in range(L)] + [w_hi[c] for c in range(L)]
        b_sc = [b_lo[c] for c in range(L)] + [b_hi[c] for c in range(L)]

        def issue_gather(j, b):
            for q in range(BCHUNK // L):
                blk_v[b, pl.ds(q * L, L)] = lax.shift_right_logical(
                    idx_v[j, pl.ds(q * L, L)], 2)
            pltpu.async_copy(
                table_hbm.at[blk_v.at[b]], rows[b], gsem[b])

        def wait_gather(b):
            pltpu.make_async_copy(
                table_hbm.at[blk_v.at[b]], rows[b], gsem[b]).wait()

        def out_slice(j):
            return out_hbm.at[j, :, pl.ds(b0, BCHUNK)]

        def issue_out(j, b):
            pltpu.async_copy(obuf[b], out_slice(j), osem[b])

        def wait_out(j, b):
            pltpu.make_async_copy(obuf[b], out_slice(j), osem[b]).wait()

        def compute(j, b):
            src, dst = rows[b], obuf[b]

            def group_body(g, _):
                t0 = g * L
                lane = lax.iota(jnp.int32, L)
                toks = t0 + lane
                sub = (idx_v[j, pl.ds(t0, L)] & 3) * D
                diag = []
                part = [jnp.zeros((L,), jnp.float32) for _ in range(4)]
                for m in range(D):
                    cl = (m + lane) & (D - 1)
                    v = plsc.load_gather(src, [toks, sub + cl])
                    diag.append(v)
                    part[m % 4] = part[m % 4] + v
                u = ((part[0] + part[1]) + (part[2] + part[3])) * (1.0 / D)
                part = [jnp.zeros((L,), jnp.float32) for _ in range(4)]
                for m in range(D):
                    diag[m] = diag[m] - u
                    part[m % 4] = part[m % 4] + diag[m] * diag[m]
                s2 = (part[0] + part[1]) + (part[2] + part[3])
                inv = _rsqrt(jnp.maximum(s2 * (1.0 / D), 0.0) + EPS)
                for m in range(D):
                    cl = (m + lane) & (D - 1)
                    plsc.store_scatter(dst, [cl, toks], diag[m] * inv)
                for c in range(D):
                    x = dst[c, pl.ds(t0, L)]
                    dst[c, pl.ds(t0, L)] = x * w_sc[c] + b_sc[c]
                return 0

            lax.fori_loop(0, BCHUNK // L, group_body, 0)

        issue_gather(0, 0)
        issue_gather(1, 1)

        for j in range(NBUF):
            b = j % NBUF
            wait_gather(b)
            compute(j, b)
            issue_out(j, b)
            issue_gather(j + NBUF, b)

        def steady(i, _):
            j0 = NBUF + i * NBUF
            for b in range(NBUF):
                j = j0 + b
                wait_gather(b)
                wait_out(j - NBUF, b)
                compute(j, b)
                issue_out(j, b)
                issue_gather(j + NBUF, b)
            return 0

        lax.fori_loop(0, (n_super - 2 * NBUF) // NBUF, steady, 0)

        for j in range(n_super - NBUF, n_super):
            b = j % NBUF
            wait_gather(b)
            wait_out(j - NBUF, b)
            compute(j, b)
            issue_out(j, b)
        for j in range(n_super - NBUF, n_super):
            wait_out(j, j % NBUF)

    out = k(ids_t, tbl_blk, ln_weight, ln_bias)
    return jnp.transpose(out, (2, 0, 1))


# speedup vs baseline: 1.5500x; 1.0154x over previous
import functools

import jax
import jax.numpy as jnp
from jax import lax
from jax.experimental import pallas as pl
from jax.experimental.pallas import tpu as pltpu
from jax.experimental.pallas import tpu_sc as plsc

EPS = 1e-12
L = 16
BCHUNK = 128
NBUF = 2
BLK = 4

_COMPILER_PARAMS = pltpu.CompilerParams(
    needs_layout_passes=False, use_tc_tiling_on_sc=True
)


def _rsqrt(x):
    xi = lax.bitcast_convert_type(x, jnp.int32)
    yi = jnp.int32(0x5F3759DF) - lax.shift_right_arithmetic(xi, 1)
    y = lax.bitcast_convert_type(yi, jnp.float32)
    for _ in range(3):
        y = y * (1.5 - 0.5 * x * y * y)
    return y


def _block_table(table, mesh):
    V, D = table.shape
    PANEL = BLK * D
    n_full = V // PANEL
    tail = V - n_full * PANEL
    n_tail = tail // BLK
    NW = 32
    tail_blk = table[n_full * PANEL:].reshape(n_tail, BLK * D)

    @functools.partial(
        pl.kernel,
        mesh=mesh,
        compiler_params=_COMPILER_PARAMS,
        out_type=jax.ShapeDtypeStruct((V // BLK, BLK * D), jnp.float32),
        scratch_types=[
            pltpu.VMEM((D, PANEL), jnp.float32),
            pltpu.VMEM((D, PANEL), jnp.float32),
            pltpu.VMEM((D, PANEL), jnp.float32),
            pltpu.VMEM((D, PANEL), jnp.float32),
            pltpu.SemaphoreType.DMA,
            pltpu.SemaphoreType.DMA,
            pltpu.SemaphoreType.DMA,
            pltpu.SemaphoreType.DMA,
        ],
    )
    def tk(tbl_t, tail_hbm, out_hbm, in0, in1, ob0, ob1, gi0, gi1, go0, go1):
        wid = lax.axis_index("s") * 2 + lax.axis_index("c")
        tbuf = [in0, in1]
        obuf = [ob0, ob1]
        isem = [gi0, gi1]
        osem = [go0, go1]
        n_mine = (n_full - wid + NW - 1) // NW

        def panel(t):
            return wid + t * NW

        def issue_in(t, b):
            pltpu.async_copy(
                tbl_t.at[:, pl.ds(panel(t) * PANEL, PANEL)], tbuf[b], isem[b])

        def wait_in(t, b):
            pltpu.make_async_copy(
                tbl_t.at[:, pl.ds(panel(t) * PANEL, PANEL)], tbuf[b],
                isem[b]).wait()

        def issue_out(t, b):
            pltpu.async_copy(
                obuf[b], out_hbm.at[pl.ds(panel(t) * D, D)], osem[b])

        def wait_out(t, b):
            pltpu.make_async_copy(
                obuf[b], out_hbm.at[pl.ds(panel(t) * D, D)], osem[b]).wait()

        def transpose(b, ngroups):
            src, dst = tbuf[b], obuf[b]
            lane = lax.iota(jnp.int32, L)
            for g in range(ngroups):
                vv = g * L + lane
                rr = lax.shift_right_logical(vv, 2)
                cb = (vv & 3) * D
                vals = [
                    plsc.load_gather(src, [(m + lane) & (D - 1), vv])
                    for m in range(D)
                ]
                for m in range(D):
                    jl = (m + lane) & (D - 1)
                    plsc.store_scatter(dst, [rr, cb + jl], vals[m])

        @pl.when(n_mine > 0)
        def _():
            issue_in(0, 0)

        @pl.when(n_mine > 1)
        def _():
            issue_in(1, 1)

        def head(t, b):
            wait_in(t, b)
            transpose(b, PANEL // L)
            issue_out(t, b)

            @pl.when(t + NBUF < n_mine)
            def _():
                issue_in(t + NBUF, b)

        @pl.when(n_mine > 0)
        def _():
            head(0, 0)

        @pl.when(n_mine > 1)
        def _():
            head(1, 1)

        def steady(i, _):
            for b in range(NBUF):
                t = NBUF + i * NBUF + b

                @pl.when(t < n_mine)
                def _():
                    wait_in(t, b)
                    wait_out(t - NBUF, b)
                    transpose(b, PANEL // L)
                    issue_out(t, b)

                    @pl.when(t + NBUF < n_mine)
                    def _():
                        issue_in(t + NBUF, b)
            return 0

        lax.fori_loop(
            0, jnp.maximum(n_mine - NBUF + 1, 0) // NBUF + 1, steady, 0)

        for b in range(NBUF):
            t_lb = jnp.where((n_mine - 1) % NBUF == b, n_mine - 1, n_mine - 2)

            @pl.when(t_lb >= 0)
            def _():
                wait_out(t_lb, b)

        @pl.when(wid == 0)
        def _():
            pltpu.sync_copy(tail_hbm, in0.at[pl.ds(0, n_tail)])
            pltpu.sync_copy(
                in0.at[pl.ds(0, n_tail)],
                out_hbm.at[pl.ds(n_full * D, n_tail)])

    return tk(table.T, tail_blk)


def kernel(input_tag_ids, table, ln_weight, ln_bias):
    B, S = input_tag_ids.shape
    V, D = table.shape
    NC, NS = 2, 16
    NW = NC * NS
    n_super = S
    assert NW * BCHUNK == B and D == 2 * L and BLK * D == 128

    ids_t = input_tag_ids.T
    mesh = plsc.VectorSubcoreMesh(core_axis_name="c", subcore_axis_name="s")
    tbl_blk = _block_table(table, mesh)
    GNBUF = 4

    @functools.partial(
        pl.kernel,
        mesh=mesh,
        compiler_params=_COMPILER_PARAMS,
        out_type=jax.ShapeDtypeStruct((S, D, B), jnp.float32),
        scratch_types=[
            pltpu.VMEM((n_super, BCHUNK), jnp.int32),
            pltpu.VMEM((GNBUF, BCHUNK), jnp.int32),
        ] + [pltpu.VMEM((BCHUNK, BLK * D), jnp.float32)] * GNBUF
        + [pltpu.VMEM((D, BCHUNK), jnp.float32)] * GNBUF
        + [
            pltpu.VMEM((D,), jnp.float32),
            pltpu.VMEM((D,), jnp.float32),
        ] + [pltpu.SemaphoreType.DMA] * (2 * GNBUF + 1),
    )
    def k(idx_hbm, table_hbm, w_hbm, b_hbm, out_hbm, idx_v, blk_v, *bufs):
        rows = list(bufs[:GNBUF])
        obuf = list(bufs[GNBUF:2 * GNBUF])
        w_v, b_v = bufs[2 * GNBUF], bufs[2 * GNBUF + 1]
        gsem = list(bufs[2 * GNBUF + 2:3 * GNBUF + 2])
        osem = list(bufs[3 * GNBUF + 2:4 * GNBUF + 2])
        isem = bufs[4 * GNBUF + 2]
        wid = lax.axis_index("s") * NC + lax.axis_index("c")
        b0 = wid * BCHUNK

        def idx_issue(t, _):
            pltpu.async_copy(
                idx_hbm.at[t, pl.ds(b0, BCHUNK)], idx_v.at[t], isem)
            return 0

        def idx_drain(t, _):
            pltpu.make_async_copy(
                idx_hbm.at[t, pl.ds(b0, BCHUNK)], idx_v.at[t], isem).wait()
            return 0

        lax.fori_loop(0, n_super, idx_issue, 0)
        pltpu.sync_copy(w_hbm, w_v)
        pltpu.sync_copy(b_hbm, b_v)
        lax.fori_loop(0, n_super, idx_drain, 0)
        w_lo, w_hi = w_v[pl.ds(0, L)], w_v[pl.ds(L, L)]
        b_lo, b_hi = b_v[pl.ds(0, L)], b_v[pl.ds(L, L)]
        w_sc = [w_lo[c] for c in range(L)] + [w_hi[c] for c in range(L)]
        b_sc = [b_lo[c] for c in range(L)] + [b_hi[c] for c in range(L)]

        def issue_gather(j, b):
            for q in range(BCHUNK // L):
                blk_v[b, pl.ds(q * L, L)] = lax.shift_right_logical(
                    idx_v[j, pl.ds(q * L, L)], 2)
            pltpu.async_copy(
                table_hbm.at[blk_v.at[b]], rows[b], gsem[b])

        def wait_gather(b):
            pltpu.make_async_copy(
                table_hbm.at[blk_v.at[b]], rows[b], gsem[b]).wait()

        def out_slice(j):
            return out_hbm.at[j, :, pl.ds(b0, BCHUNK)]

        def issue_out(j, b):
            pltpu.async_copy(obuf[b], out_slice(j), osem[b])

        def wait_out(j, b):
            pltpu.make_async_copy(obuf[b], out_slice(j), osem[b]).wait()

        def compute(j, b):
            src, dst = rows[b], obuf[b]

            def group_body(g, _):
                t0 = g * L
                lane = lax.iota(jnp.int32, L)
                toks = t0 + lane
                sub = (idx_v[j, pl.ds(t0, L)] & 3) * D
                diag = []
                part = [jnp.zeros((L,), jnp.float32) for _ in range(4)]
                for m in range(D):
                    cl = (m + lane) & (D - 1)
                    v = plsc.load_gather(src, [toks, sub + cl])
                    diag.append(v)
                    part[m % 4] = part[m % 4] + v
                u = ((part[0] + part[1]) + (part[2] + part[3])) * (1.0 / D)
                part = [jnp.zeros((L,), jnp.float32) for _ in range(4)]
                for m in range(D):
                    diag[m] = diag[m] - u
                    part[m % 4] = part[m % 4] + diag[m] * diag[m]
                s2 = (part[0] + part[1]) + (part[2] + part[3])
                inv = _rsqrt(jnp.maximum(s2 * (1.0 / D), 0.0) + EPS)
                for m in range(D):
                    cl = (m + lane) & (D - 1)
                    plsc.store_scatter(dst, [cl, toks], diag[m] * inv)
                for c in range(D):
                    x = dst[c, pl.ds(t0, L)]
                    dst[c, pl.ds(t0, L)] = x * w_sc[c] + b_sc[c]
                return 0

            lax.fori_loop(0, BCHUNK // L, group_body, 0)

        for j in range(GNBUF):
            issue_gather(j, j)

        for j in range(GNBUF):
            b = j % GNBUF
            wait_gather(b)
            compute(j, b)
            issue_out(j, b)
            issue_gather(j + GNBUF, b)

        def steady(i, _):
            j0 = GNBUF + i * GNBUF
            for b in range(GNBUF):
                j = j0 + b
                wait_gather(b)
                wait_out(j - GNBUF, b)
                compute(j, b)
                issue_out(j, b)
                issue_gather(j + GNBUF, b)
            return 0

        lax.fori_loop(0, (n_super - 2 * GNBUF) // GNBUF, steady, 0)

        for j in range(n_super - GNBUF, n_super):
            b = j % GNBUF
            wait_gather(b)
            wait_out(j - GNBUF, b)
            compute(j, b)
            issue_out(j, b)
        for j in range(n_super - GNBUF, n_super):
            wait_out(j, j % GNBUF)

    out = k(ids_t, tbl_blk, ln_weight, ln_bias)
    return jnp.transpose(out, (2, 0, 1))
